# Initial kernel scaffold; baseline (speedup 1.0000x reference)
import jax
import jax.numpy as jnp
from jax.experimental import pallas as pl


def kernel(x, edge_index, batch, W1, b1, W2, b2, W3, b3, lin_W, lin_b):
    raise NotImplementedError("write your pallas kernel here")


# speedup vs baseline: 19.3621x; 19.3621x over previous
import functools

import jax
import jax.numpy as jnp
from jax import lax
from jax.experimental import pallas as pl
from jax.experimental.pallas import tpu as pltpu
from jax.experimental.pallas import tpu_sc as plsc

N = 100000
E = 3200000
NUM_GRAPHS = 128
HID = 64

NC = 2
NS = 16
NW = NC * NS
EPW = E // NW
CHUNK = 1000
NPAD = 100096
RPT = NPAD // NS

BN = 2000
GRID = N // BN

_mesh = plsc.VectorSubcoreMesh(core_axis_name="c", subcore_axis_name="s")


@functools.partial(
    pl.kernel,
    mesh=_mesh,
    compiler_params=pltpu.CompilerParams(use_tc_tiling_on_sc=False),
    out_type=jax.ShapeDtypeStruct((NC, NPAD, 16), jnp.float32),
    scratch_types=[
        pltpu.VMEM((CHUNK,), jnp.int32),
        pltpu.VMEM((CHUNK, 16), jnp.float32),
        pltpu.VMEM_SHARED((NPAD, 16), jnp.float32),
    ],
)
def _sc_degree(dst_hbm, ones_hbm, z_hbm, out_hbm, dst_v, ones_v, acc_sh):
    c = lax.axis_index("c")
    s = lax.axis_index("s")
    wid = c * NS + s
    ebase = wid * EPW
    pltpu.sync_copy(z_hbm.at[pl.ds(s * RPT, RPT)], acc_sh.at[pl.ds(s * RPT, RPT)])
    pltpu.sync_copy(ones_hbm, ones_v)
    plsc.subcore_barrier()

    def chunk_body(i, carry):
        pltpu.sync_copy(dst_hbm.at[pl.ds(ebase + i * CHUNK, CHUNK)], dst_v)
        pltpu.sync_copy(ones_v, acc_sh.at[dst_v], add=True)
        return carry

    lax.fori_loop(0, EPW // CHUNK, chunk_body, 0)
    plsc.subcore_barrier()
    pltpu.sync_copy(acc_sh.at[pl.ds(s * RPT, RPT)],
                    out_hbm.at[c, pl.ds(s * RPT, RPT)])


def _make_sc_agg(num_passes):
    @functools.partial(
        pl.kernel,
        mesh=_mesh,
        compiler_params=pltpu.CompilerParams(use_tc_tiling_on_sc=False),
        out_type=[jax.ShapeDtypeStruct((NC, NPAD, 16), jnp.float32)] * num_passes,
        scratch_types=[
            pltpu.VMEM((CHUNK,), jnp.int32),
            pltpu.VMEM((CHUNK,), jnp.int32),
            pltpu.VMEM((CHUNK, 16), jnp.float32),
            pltpu.VMEM_SHARED((NPAD, 16), jnp.float32),
            pltpu.SemaphoreType.DMA,
        ],
    )
    def sc_agg(src_hbm, dst_hbm, *rest):
        tables = rest[:num_passes]
        z_hbm = rest[num_passes]
        outs = rest[num_passes + 1:2 * num_passes + 1]
        src_v, dst_v, rows_v, acc_sh, sem = rest[2 * num_passes + 1:]
        c = lax.axis_index("c")
        s = lax.axis_index("s")
        wid = c * NS + s
        ebase = wid * EPW
        for k in range(num_passes):
            pltpu.sync_copy(z_hbm.at[pl.ds(s * RPT, RPT)],
                            acc_sh.at[pl.ds(s * RPT, RPT)])
            plsc.subcore_barrier()

            def chunk_body(i, carry):
                off = ebase + i * CHUNK
                pltpu.sync_copy(src_hbm.at[pl.ds(off, CHUNK)], src_v)
                pltpu.sync_copy(dst_hbm.at[pl.ds(off, CHUNK)], dst_v)
                pltpu.async_copy(tables[k].at[src_v], rows_v, sem).wait()
                pltpu.sync_copy(rows_v, acc_sh.at[dst_v], add=True)
                return carry

            lax.fori_loop(0, EPW // CHUNK, chunk_body, 0)
            plsc.subcore_barrier()
            pltpu.sync_copy(acc_sh.at[pl.ds(s * RPT, RPT)],
                            outs[k].at[c, pl.ds(s * RPT, RPT)])
            plsc.subcore_barrier()

    return sc_agg

_sc_agg1 = _make_sc_agg(1)
_sc_agg4 = _make_sc_agg(4)


def _tc_prep_body(degp_ref, x_ref, dis_ref, hp1_ref):
    deg = degp_ref[0, :, 0:1] + degp_ref[1, :, 0:1] + 1.0
    dis = lax.rsqrt(deg)
    dis_ref[...] = dis
    hp1_ref[...] = dis * x_ref[...]


def _tc_prep(degp, xpad):
    return pl.pallas_call(
        _tc_prep_body,
        grid=(GRID,),
        in_specs=[
            pl.BlockSpec((NC, BN, 16), lambda i: (0, i, 0)),
            pl.BlockSpec((BN, 16), lambda i: (i, 0)),
        ],
        out_specs=[
            pl.BlockSpec((BN, 1), lambda i: (i, 0)),
            pl.BlockSpec((BN, 16), lambda i: (i, 0)),
        ],
        out_shape=[
            jax.ShapeDtypeStruct((N, 1), jnp.float32),
            jax.ShapeDtypeStruct((N, 16), jnp.float32),
        ],
    )(degp, xpad)


def _tc_dense1_body(p_ref, hp1_ref, dis_ref, w1_ref, b1_ref, w2_ref,
                    o0, o1, o2, o3):
    dis = dis_ref[...]
    agg16 = dis * (p_ref[0] + p_ref[1] + hp1_ref[...])
    out1 = jnp.dot(agg16, w1_ref[...],
                   preferred_element_type=jnp.float32) + b1_ref[...]
    h2 = jnp.maximum(out1, 0.0)
    hp2 = dis * jnp.dot(h2, w2_ref[...], preferred_element_type=jnp.float32)
    o0[...] = hp2[:, 0:16]
    o1[...] = hp2[:, 16:32]
    o2[...] = hp2[:, 32:48]
    o3[...] = hp2[:, 48:64]


def _tc_dense1(p, hp1, dis, w1pad, b1, w2):
    return pl.pallas_call(
        _tc_dense1_body,
        grid=(GRID,),
        in_specs=[
            pl.BlockSpec((NC, BN, 16), lambda i: (0, i, 0)),
            pl.BlockSpec((BN, 16), lambda i: (i, 0)),
            pl.BlockSpec((BN, 1), lambda i: (i, 0)),
            pl.BlockSpec((16, HID), lambda i: (0, 0)),
            pl.BlockSpec((1, HID), lambda i: (0, 0)),
            pl.BlockSpec((HID, HID), lambda i: (0, 0)),
        ],
        out_specs=[pl.BlockSpec((BN, 16), lambda i: (i, 0))] * 4,
        out_shape=[jax.ShapeDtypeStruct((N, 16), jnp.float32)] * 4,
    )(p, hp1, dis, w1pad, b1, w2)


def _tc_dense2_body(p0, p1, p2, p3, s0, s1, s2, s3, dis_ref, b_ref, w_ref,
                    o0, o1, o2, o3):
    dis = dis_ref[...]
    cols = [p[0] + p[1] + s[...] for p, s in ((p0, s0), (p1, s1),
                                              (p2, s2), (p3, s3))]
    agg = jnp.concatenate(cols, axis=1)
    out = dis * agg + b_ref[...]
    h = jnp.maximum(out, 0.0)
    hp = dis * jnp.dot(h, w_ref[...], preferred_element_type=jnp.float32)
    o0[...] = hp[:, 0:16]
    o1[...] = hp[:, 16:32]
    o2[...] = hp[:, 32:48]
    o3[...] = hp[:, 48:64]


def _tc_dense2(parts, selfs, dis, b, w):
    return pl.pallas_call(
        _tc_dense2_body,
        grid=(GRID,),
        in_specs=(
            [pl.BlockSpec((NC, BN, 16), lambda i: (0, i, 0))] * 4
            + [pl.BlockSpec((BN, 16), lambda i: (i, 0))] * 4
            + [
                pl.BlockSpec((BN, 1), lambda i: (i, 0)),
                pl.BlockSpec((1, HID), lambda i: (0, 0)),
                pl.BlockSpec((HID, HID), lambda i: (0, 0)),
            ]
        ),
        out_specs=[pl.BlockSpec((BN, 16), lambda i: (i, 0))] * 4,
        out_shape=[jax.ShapeDtypeStruct((N, 16), jnp.float32)] * 4,
    )(*parts, *selfs, dis, b, w)


def _tc_final_body(p0, p1, p2, p3, s0, s1, s2, s3, dis_ref, b_ref,
                   batch_ref, lw_ref, lb_ref, out_ref, acc, cnt):
    i = pl.program_id(0)

    @pl.when(i == 0)
    def _init():
        acc[...] = jnp.zeros_like(acc)
        cnt[...] = jnp.zeros_like(cnt)

    dis = dis_ref[...]
    cols = [p[0] + p[1] + s[...] for p, s in ((p0, s0), (p1, s1),
                                              (p2, s2), (p3, s3))]
    agg = jnp.concatenate(cols, axis=1)
    out3 = dis * agg + b_ref[...]

    b_ids = batch_ref[0, 0, :]
    gids = lax.broadcasted_iota(jnp.int32, (BN, NUM_GRAPHS), 1)
    onehot = (b_ids[:, None] == gids).astype(jnp.float32)
    acc[...] += lax.dot_general(onehot, out3, (((0,), (0,)), ((), ())),
                                preferred_element_type=jnp.float32)
    cnt[...] += jnp.sum(onehot, axis=0)[:, None]

    @pl.when(i == GRID - 1)
    def _fin():
        pooled = acc[...] / jnp.maximum(cnt[...], 1.0)
        out_ref[...] = jnp.dot(pooled, lw_ref[...],
                               preferred_element_type=jnp.float32) + lb_ref[...]


def _tc_final(parts, selfs, dis, b, batch3d, lin_W, lin_b):
    return pl.pallas_call(
        _tc_final_body,
        grid=(GRID,),
        in_specs=(
            [pl.BlockSpec((NC, BN, 16), lambda i: (0, i, 0))] * 4
            + [pl.BlockSpec((BN, 16), lambda i: (i, 0))] * 4
            + [
                pl.BlockSpec((BN, 1), lambda i: (i, 0)),
                pl.BlockSpec((1, HID), lambda i: (0, 0)),
                pl.BlockSpec((1, 1, BN), lambda i: (i, 0, 0)),
                pl.BlockSpec((HID, 3), lambda i: (0, 0)),
                pl.BlockSpec((1, 3), lambda i: (0, 0)),
            ]
        ),
        out_specs=pl.BlockSpec((NUM_GRAPHS, 3), lambda i: (0, 0)),
        out_shape=jax.ShapeDtypeStruct((NUM_GRAPHS, 3), jnp.float32),
        scratch_shapes=[
            pltpu.VMEM((NUM_GRAPHS, HID), jnp.float32),
            pltpu.VMEM((NUM_GRAPHS, 1), jnp.float32),
        ],
        compiler_params=pltpu.CompilerParams(
            dimension_semantics=("arbitrary",)),
    )(*parts, *selfs, dis, b, batch3d, lin_W, lin_b)


def kernel(x, edge_index, batch, W1, b1, W2, b2, W3, b3, lin_W, lin_b):
    src = edge_index[0]
    dst = edge_index[1]

    zeros16 = jnp.zeros((NPAD, 16), jnp.float32)
    ones_c = jnp.ones((CHUNK, 16), jnp.float32)

    degp = _sc_degree(dst, ones_c, zeros16)

    xpad = jnp.pad(x, ((0, 0), (0, 12)))
    dis, hp1 = _tc_prep(degp, xpad)

    (p1,) = _sc_agg1(src, dst, hp1, zeros16)

    w1pad = jnp.pad(W1, ((0, 12), (0, 0)))
    hp2 = _tc_dense1(p1, hp1, dis, w1pad, b1[None, :], W2)

    p2 = _sc_agg4(src, dst, *hp2, zeros16)
    hp3 = _tc_dense2(p2, hp2, dis, b2[None, :], W3)

    p3 = _sc_agg4(src, dst, *hp3, zeros16)
    batch3d = batch.reshape(GRID, 1, BN)
    return _tc_final(p3, hp3, dis, b3[None, :], batch3d, lin_W, lin_b[None, :])


# speedup vs baseline: 25.5642x; 1.3203x over previous
import functools

import jax
import jax.numpy as jnp
from jax import lax
from jax.experimental import pallas as pl
from jax.experimental.pallas import tpu as pltpu
from jax.experimental.pallas import tpu_sc as plsc

N = 100000
E = 3200000
NUM_GRAPHS = 128
HID = 64

NC = 2
NS = 16
NW = NC * NS
EPW = E // NW
CHUNK = 800
NPAD = 100096
RPT = NPAD // NS

BN = 2000
GRID = N // BN

_mesh = plsc.VectorSubcoreMesh(core_axis_name="c", subcore_axis_name="s")


@functools.partial(
    pl.kernel,
    mesh=_mesh,
    compiler_params=pltpu.CompilerParams(use_tc_tiling_on_sc=False),
    out_type=jax.ShapeDtypeStruct((NC, NPAD, 16), jnp.float32),
    scratch_types=[
        pltpu.VMEM((CHUNK,), jnp.int32),
        pltpu.VMEM((CHUNK, 16), jnp.float32),
        pltpu.VMEM_SHARED((NPAD, 16), jnp.float32),
    ],
)
def _sc_degree(dst_hbm, ones_hbm, z_hbm, out_hbm, dst_v, ones_v, acc_sh):
    c = lax.axis_index("c")
    s = lax.axis_index("s")
    wid = c * NS + s
    ebase = wid * EPW
    pltpu.sync_copy(z_hbm.at[pl.ds(s * RPT, RPT)], acc_sh.at[pl.ds(s * RPT, RPT)])
    pltpu.sync_copy(ones_hbm, ones_v)
    plsc.subcore_barrier()

    def chunk_body(i, carry):
        pltpu.sync_copy(dst_hbm.at[pl.ds(ebase + i * CHUNK, CHUNK)], dst_v)
        pltpu.sync_copy(ones_v, acc_sh.at[dst_v], add=True)
        return carry

    lax.fori_loop(0, EPW // CHUNK, chunk_body, 0)
    plsc.subcore_barrier()
    pltpu.sync_copy(acc_sh.at[pl.ds(s * RPT, RPT)],
                    out_hbm.at[c, pl.ds(s * RPT, RPT)])


def _make_sc_agg(num_passes):
    @functools.partial(
        pl.kernel,
        mesh=_mesh,
        compiler_params=pltpu.CompilerParams(use_tc_tiling_on_sc=False),
        out_type=[jax.ShapeDtypeStruct((NC, NPAD, 16), jnp.float32)] * num_passes,
        scratch_types=[
            pltpu.VMEM((CHUNK,), jnp.int32),
            pltpu.VMEM((CHUNK,), jnp.int32),
            pltpu.VMEM((CHUNK,), jnp.int32),
            pltpu.VMEM((CHUNK,), jnp.int32),
            pltpu.VMEM((CHUNK, 16), jnp.float32),
            pltpu.VMEM((CHUNK, 16), jnp.float32),
            pltpu.VMEM_SHARED((NPAD, 16), jnp.float32),
            pltpu.SemaphoreType.DMA,
            pltpu.SemaphoreType.DMA,
        ],
    )
    def sc_agg(src_hbm, dst_hbm, *rest):
        tables = rest[:num_passes]
        z_hbm = rest[num_passes]
        outs = rest[num_passes + 1:2 * num_passes + 1]
        (src0, src1, dst0, dst1, rows0, rows1,
         acc_sh, sem0, sem1) = rest[2 * num_passes + 1:]
        c = lax.axis_index("c")
        s = lax.axis_index("s")
        wid = c * NS + s
        ebase = wid * EPW
        nch = EPW // CHUNK
        for k in range(num_passes):
            tab = tables[k]
            pltpu.sync_copy(z_hbm.at[pl.ds(s * RPT, RPT)],
                            acc_sh.at[pl.ds(s * RPT, RPT)])
            plsc.subcore_barrier()

            pltpu.sync_copy(src_hbm.at[pl.ds(ebase, CHUNK)], src0)
            pltpu.sync_copy(dst_hbm.at[pl.ds(ebase, CHUNK)], dst0)
            pltpu.async_copy(tab.at[src0], rows0, sem0)

            def pair_body(g, carry):
                off1 = ebase + (2 * g + 1) * CHUNK
                pltpu.sync_copy(src_hbm.at[pl.ds(off1, CHUNK)], src1)
                pltpu.sync_copy(dst_hbm.at[pl.ds(off1, CHUNK)], dst1)
                pltpu.async_copy(tab.at[src1], rows1, sem1)
                pltpu.make_async_copy(tab.at[src0], rows0, sem0).wait()
                pltpu.sync_copy(rows0, acc_sh.at[dst0], add=True)

                off2 = ebase + (2 * g + 2) * CHUNK
                pltpu.sync_copy(src_hbm.at[pl.ds(off2, CHUNK)], src0)
                pltpu.sync_copy(dst_hbm.at[pl.ds(off2, CHUNK)], dst0)
                pltpu.async_copy(tab.at[src0], rows0, sem0)
                pltpu.make_async_copy(tab.at[src1], rows1, sem1).wait()
                pltpu.sync_copy(rows1, acc_sh.at[dst1], add=True)
                return carry

            lax.fori_loop(0, (nch - 1) // 2, pair_body, 0)
            pltpu.make_async_copy(tab.at[src0], rows0, sem0).wait()
            pltpu.sync_copy(rows0, acc_sh.at[dst0], add=True)
            plsc.subcore_barrier()
            pltpu.sync_copy(acc_sh.at[pl.ds(s * RPT, RPT)],
                            outs[k].at[c, pl.ds(s * RPT, RPT)])
            plsc.subcore_barrier()

    return sc_agg

_sc_agg1 = _make_sc_agg(1)
_sc_agg4 = _make_sc_agg(4)


def _tc_prep_body(degp_ref, x_ref, dis_ref, hp1_ref):
    deg = degp_ref[0, :, 0:1] + degp_ref[1, :, 0:1] + 1.0
    dis = lax.rsqrt(deg)
    dis_ref[...] = dis
    hp1_ref[...] = dis * x_ref[...]


def _tc_prep(degp, xpad):
    return pl.pallas_call(
        _tc_prep_body,
        grid=(GRID,),
        in_specs=[
            pl.BlockSpec((NC, BN, 16), lambda i: (0, i, 0)),
            pl.BlockSpec((BN, 16), lambda i: (i, 0)),
        ],
        out_specs=[
            pl.BlockSpec((BN, 1), lambda i: (i, 0)),
            pl.BlockSpec((BN, 16), lambda i: (i, 0)),
        ],
        out_shape=[
            jax.ShapeDtypeStruct((N, 1), jnp.float32),
            jax.ShapeDtypeStruct((N, 16), jnp.float32),
        ],
    )(degp, xpad)


def _tc_dense1_body(p_ref, hp1_ref, dis_ref, w1_ref, b1_ref, w2_ref,
                    o0, o1, o2, o3):
    dis = dis_ref[...]
    agg16 = dis * (p_ref[0] + p_ref[1] + hp1_ref[...])
    out1 = jnp.dot(agg16, w1_ref[...],
                   preferred_element_type=jnp.float32) + b1_ref[...]
    h2 = jnp.maximum(out1, 0.0)
    hp2 = dis * jnp.dot(h2, w2_ref[...], preferred_element_type=jnp.float32)
    o0[...] = hp2[:, 0:16]
    o1[...] = hp2[:, 16:32]
    o2[...] = hp2[:, 32:48]
    o3[...] = hp2[:, 48:64]


def _tc_dense1(p, hp1, dis, w1pad, b1, w2):
    return pl.pallas_call(
        _tc_dense1_body,
        grid=(GRID,),
        in_specs=[
            pl.BlockSpec((NC, BN, 16), lambda i: (0, i, 0)),
            pl.BlockSpec((BN, 16), lambda i: (i, 0)),
            pl.BlockSpec((BN, 1), lambda i: (i, 0)),
            pl.BlockSpec((16, HID), lambda i: (0, 0)),
            pl.BlockSpec((1, HID), lambda i: (0, 0)),
            pl.BlockSpec((HID, HID), lambda i: (0, 0)),
        ],
        out_specs=[pl.BlockSpec((BN, 16), lambda i: (i, 0))] * 4,
        out_shape=[jax.ShapeDtypeStruct((N, 16), jnp.float32)] * 4,
    )(p, hp1, dis, w1pad, b1, w2)


def _tc_dense2_body(p0, p1, p2, p3, s0, s1, s2, s3, dis_ref, b_ref, w_ref,
                    o0, o1, o2, o3):
    dis = dis_ref[...]
    cols = [p[0] + p[1] + s[...] for p, s in ((p0, s0), (p1, s1),
                                              (p2, s2), (p3, s3))]
    agg = jnp.concatenate(cols, axis=1)
    out = dis * agg + b_ref[...]
    h = jnp.maximum(out, 0.0)
    hp = dis * jnp.dot(h, w_ref[...], preferred_element_type=jnp.float32)
    o0[...] = hp[:, 0:16]
    o1[...] = hp[:, 16:32]
    o2[...] = hp[:, 32:48]
    o3[...] = hp[:, 48:64]


def _tc_dense2(parts, selfs, dis, b, w):
    return pl.pallas_call(
        _tc_dense2_body,
        grid=(GRID,),
        in_specs=(
            [pl.BlockSpec((NC, BN, 16), lambda i: (0, i, 0))] * 4
            + [pl.BlockSpec((BN, 16), lambda i: (i, 0))] * 4
            + [
                pl.BlockSpec((BN, 1), lambda i: (i, 0)),
                pl.BlockSpec((1, HID), lambda i: (0, 0)),
                pl.BlockSpec((HID, HID), lambda i: (0, 0)),
            ]
        ),
        out_specs=[pl.BlockSpec((BN, 16), lambda i: (i, 0))] * 4,
        out_shape=[jax.ShapeDtypeStruct((N, 16), jnp.float32)] * 4,
    )(*parts, *selfs, dis, b, w)


def _tc_final_body(p0, p1, p2, p3, s0, s1, s2, s3, dis_ref, b_ref,
                   batch_ref, lw_ref, lb_ref, out_ref, acc, cnt):
    i = pl.program_id(0)

    @pl.when(i == 0)
    def _init():
        acc[...] = jnp.zeros_like(acc)
        cnt[...] = jnp.zeros_like(cnt)

    dis = dis_ref[...]
    cols = [p[0] + p[1] + s[...] for p, s in ((p0, s0), (p1, s1),
                                              (p2, s2), (p3, s3))]
    agg = jnp.concatenate(cols, axis=1)
    out3 = dis * agg + b_ref[...]

    b_ids = batch_ref[0, 0, :]
    gids = lax.broadcasted_iota(jnp.int32, (BN, NUM_GRAPHS), 1)
    onehot = (b_ids[:, None] == gids).astype(jnp.float32)
    acc[...] += lax.dot_general(onehot, out3, (((0,), (0,)), ((), ())),
                                preferred_element_type=jnp.float32)
    cnt[...] += jnp.sum(onehot, axis=0)[:, None]

    @pl.when(i == GRID - 1)
    def _fin():
        pooled = acc[...] / jnp.maximum(cnt[...], 1.0)
        out_ref[...] = jnp.dot(pooled, lw_ref[...],
                               preferred_element_type=jnp.float32) + lb_ref[...]


def _tc_final(parts, selfs, dis, b, batch3d, lin_W, lin_b):
    return pl.pallas_call(
        _tc_final_body,
        grid=(GRID,),
        in_specs=(
            [pl.BlockSpec((NC, BN, 16), lambda i: (0, i, 0))] * 4
            + [pl.BlockSpec((BN, 16), lambda i: (i, 0))] * 4
            + [
                pl.BlockSpec((BN, 1), lambda i: (i, 0)),
                pl.BlockSpec((1, HID), lambda i: (0, 0)),
                pl.BlockSpec((1, 1, BN), lambda i: (i, 0, 0)),
                pl.BlockSpec((HID, 3), lambda i: (0, 0)),
                pl.BlockSpec((1, 3), lambda i: (0, 0)),
            ]
        ),
        out_specs=pl.BlockSpec((NUM_GRAPHS, 3), lambda i: (0, 0)),
        out_shape=jax.ShapeDtypeStruct((NUM_GRAPHS, 3), jnp.float32),
        scratch_shapes=[
            pltpu.VMEM((NUM_GRAPHS, HID), jnp.float32),
            pltpu.VMEM((NUM_GRAPHS, 1), jnp.float32),
        ],
        compiler_params=pltpu.CompilerParams(
            dimension_semantics=("arbitrary",)),
    )(*parts, *selfs, dis, b, batch3d, lin_W, lin_b)


def kernel(x, edge_index, batch, W1, b1, W2, b2, W3, b3, lin_W, lin_b):
    src = edge_index[0]
    dst = edge_index[1]

    zeros16 = jnp.zeros((NPAD, 16), jnp.float32)
    ones_c = jnp.ones((CHUNK, 16), jnp.float32)

    degp = _sc_degree(dst, ones_c, zeros16)

    xpad = jnp.pad(x, ((0, 0), (0, 12)))
    dis, hp1 = _tc_prep(degp, xpad)

    (p1,) = _sc_agg1(src, dst, hp1, zeros16)

    w1pad = jnp.pad(W1, ((0, 12), (0, 0)))
    hp2 = _tc_dense1(p1, hp1, dis, w1pad, b1[None, :], W2)

    p2 = _sc_agg4(src, dst, *hp2, zeros16)
    hp3 = _tc_dense2(p2, hp2, dis, b2[None, :], W3)

    p3 = _sc_agg4(src, dst, *hp3, zeros16)
    batch3d = batch.reshape(GRID, 1, BN)
    return _tc_final(p3, hp3, dis, b3[None, :], batch3d, lin_W, lin_b[None, :])
